# baseline (speedup 1.0000x reference)
import jax
import jax.numpy as jnp
from jax import lax
from jax.experimental import pallas as pl
from jax.experimental.pallas import tpu as pltpu
from jax.experimental.pallas import tpu_sc as plsc

N_NODES = 10000
N_EDGES = 320000
CHUNK = 128
NC = 2
NS = 16
E_PAD = 327680
N_ROWS = E_PAD // CHUNK
TRASH = N_NODES
N_TRASH = 128
N_ACC = 10128
BLK = 1000
GRID = N_NODES // BLK

_VMESH = plsc.VectorSubcoreMesh(
    core_axis_name="c", subcore_axis_name="s", num_cores=NC, num_subcores=NS
)

R_MAIN = 632
R_LAST = N_NODES - 15 * R_MAIN

IB = 16


def _copy_rows(src_at, dst_at, s):
  @pl.when(s < NS - 1)
  def _():
    r0 = s * R_MAIN
    pltpu.sync_copy(src_at.at[pl.ds(r0, R_MAIN)], dst_at.at[pl.ds(r0, R_MAIN)])

  @pl.when(s == NS - 1)
  def _():
    r0 = (NS - 1) * R_MAIN
    pltpu.sync_copy(src_at.at[pl.ds(r0, R_LAST)], dst_at.at[pl.ds(r0, R_LAST)])


def _gather_scatter_block(g_at, accum, src_v, dst_v, bufs, sems):
  descs = [None, None]
  descs[0] = pltpu.async_copy(g_at.at[src_v.at[0]], bufs[0], sems[0])
  for k in range(IB):
    if k + 1 < IB:
      nxt = (k + 1) % 2
      descs[nxt] = pltpu.async_copy(g_at.at[src_v.at[k + 1]], bufs[nxt],
                                    sems[nxt])
    descs[k % 2].wait()
    pltpu.sync_copy(bufs[k % 2], accum.at[dst_v.at[k]], add=True)


def _deg_hist(dst_rows, ones_hbm):
  rows_pt = N_ROWS // (NC * NS)

  def body(dst_hbm, ones_r, hist_hbm, accum, idx_v, ones_v):
    c = lax.axis_index("c")
    s = lax.axis_index("s")
    _copy_rows(ones_r, accum, s)
    pltpu.sync_copy(ones_r.at[pl.ds(0, CHUNK)], ones_v)
    plsc.subcore_barrier()

    @pl.loop(0, rows_pt // IB)
    def _(j):
      row0 = (c * NS + s) * rows_pt + j * IB
      pltpu.sync_copy(dst_hbm.at[pl.ds(row0, IB)], idx_v)

      @pl.loop(0, IB)
      def _(k):
        pltpu.sync_copy(ones_v, accum.at[idx_v.at[k]], add=True)

    plsc.subcore_barrier()
    _copy_rows(accum, hist_hbm.at[c], s)

  kern = pl.kernel(
      body,
      out_type=jax.ShapeDtypeStruct((NC, N_NODES, 16), jnp.float32),
      mesh=_VMESH,
      scratch_types=[
          pltpu.VMEM_SHARED((N_ACC, 16), jnp.float32),
          pltpu.VMEM((IB, CHUNK), jnp.int32),
          pltpu.VMEM((CHUNK, 16), jnp.float32),
      ],
  )
  return kern(dst_rows, ones_hbm)


def _agg_fsplit(g2, src_rows, dst_rows):
  rows_pt = N_ROWS // NS

  def body(g_hbm, src_hbm, dst_hbm, agg_hbm, accum,
           src_v, dst_v, rv0, rv1, sem0, sem1):
    c = lax.axis_index("c")
    s = lax.axis_index("s")
    _copy_rows(g_hbm.at[c], accum, s)
    plsc.subcore_barrier()

    @pl.loop(0, rows_pt // IB)
    def _(j):
      row0 = s * rows_pt + j * IB
      pltpu.sync_copy(src_hbm.at[pl.ds(row0, IB)], src_v)
      pltpu.sync_copy(dst_hbm.at[pl.ds(row0, IB)], dst_v)
      _gather_scatter_block(g_hbm.at[c], accum, src_v, dst_v,
                            (rv0, rv1), (sem0, sem1))

    plsc.subcore_barrier()
    _copy_rows(accum, agg_hbm.at[c], s)

  kern = pl.kernel(
      body,
      out_type=jax.ShapeDtypeStruct((NC, N_NODES, 128), jnp.float32),
      mesh=_VMESH,
      scratch_types=[
          pltpu.VMEM_SHARED((N_ACC, 128), jnp.float32),
          pltpu.VMEM((IB, CHUNK), jnp.int32),
          pltpu.VMEM((IB, CHUNK), jnp.int32),
          pltpu.VMEM((CHUNK, 128), jnp.float32),
          pltpu.VMEM((CHUNK, 128), jnp.float32),
          pltpu.SemaphoreType.DMA,
          pltpu.SemaphoreType.DMA,
      ],
  )
  return kern(g2, src_rows, dst_rows)


def _agg_esplit(g, src_rows, dst_rows):
  rows_pt = N_ROWS // (NC * NS)

  def body(g_hbm, src_hbm, dst_hbm, p_hbm, accum,
           src_v, dst_v, rv0, rv1, sem0, sem1):
    c = lax.axis_index("c")
    s = lax.axis_index("s")
    _copy_rows(g_hbm.at[c], accum, s)
    plsc.subcore_barrier()

    @pl.loop(0, rows_pt // IB)
    def _(j):
      row0 = (c * NS + s) * rows_pt + j * IB
      pltpu.sync_copy(src_hbm.at[pl.ds(row0, IB)], src_v)
      pltpu.sync_copy(dst_hbm.at[pl.ds(row0, IB)], dst_v)
      _gather_scatter_block(g_hbm.at[c], accum, src_v, dst_v,
                            (rv0, rv1), (sem0, sem1))

    plsc.subcore_barrier()
    _copy_rows(accum, p_hbm.at[c], s)

  kern = pl.kernel(
      body,
      out_type=jax.ShapeDtypeStruct((NC, N_NODES, 128), jnp.float32),
      mesh=_VMESH,
      scratch_types=[
          pltpu.VMEM_SHARED((N_ACC, 128), jnp.float32),
          pltpu.VMEM((IB, CHUNK), jnp.int32),
          pltpu.VMEM((IB, CHUNK), jnp.int32),
          pltpu.VMEM((CHUNK, 128), jnp.float32),
          pltpu.VMEM((CHUNK, 128), jnp.float32),
          pltpu.SemaphoreType.DMA,
          pltpu.SemaphoreType.DMA,
      ],
  )
  return kern(g, src_rows, dst_rows)


def _dinv(hist_ref):
  deg = hist_ref[0, :, 0:1] + hist_ref[1, :, 0:1] - 1.0
  return lax.rsqrt(deg)


def _k0_body(hist_ref, x_ref, gx_ref):
  gx = x_ref[...] * _dinv(hist_ref)
  gx_ref[0, :, :] = gx
  gx_ref[1, :, :] = gx


def _k1_body(hist_ref, px_ref, x_ref, w_ref, b_ref, g1_ref):
  dinv = _dinv(hist_ref)
  a = (px_ref[0] + px_ref[1] - x_ref[...] * dinv) * dinv
  h = jnp.dot(a, w_ref[...], preferred_element_type=jnp.float32) + b_ref[...]
  h = jnp.maximum(h, 0.0)
  g1_ref[0, :, :] = h[:, :128] * dinv
  g1_ref[1, :, :] = h[:, 128:] * dinv


def _k2_body(hist_ref, agg_ref, w2_ref, b2_ref, w3_ref, g3_ref):
  dinv = _dinv(hist_ref)
  a = jnp.concatenate([agg_ref[0], agg_ref[1]], axis=1) * dinv
  h2 = jnp.dot(a, w2_ref[...], preferred_element_type=jnp.float32) + b2_ref[...]
  h2 = jnp.maximum(h2, 0.0)
  m = jnp.dot(h2, w3_ref[...], preferred_element_type=jnp.float32) * dinv
  g3 = jnp.concatenate([m, jnp.zeros((BLK, 128 - 40), jnp.float32)], axis=1)
  g3_ref[0, :, :] = g3
  g3_ref[1, :, :] = g3


def _k3_body(hist_ref, p3_ref, g3_ref, b3_ref, out_ref):
  dinv = _dinv(hist_ref)
  a = (p3_ref[0] + p3_ref[1] - g3_ref[0]) * dinv
  out_ref[...] = a[:, :40] + b3_ref[...]


def _hist_spec():
  return pl.BlockSpec((NC, BLK, 16), lambda i: (0, i, 0))


def _split_spec(f2):
  return pl.BlockSpec((NC, BLK, f2), lambda i: (0, i, 0))


def _row_spec(width):
  return pl.BlockSpec((BLK, width), lambda i: (i, 0))


def _full(shape):
  return pl.BlockSpec(shape, lambda i: tuple(0 for _ in shape))


def kernel(x, edge_index, W1, b1, W2, b2, W3, b3):
  ei = edge_index.astype(jnp.int32)
  pad = E_PAD - N_EDGES
  src_rows = jnp.concatenate(
      [ei[0], jnp.zeros((pad,), jnp.int32)]).reshape(N_ROWS, CHUNK)
  dst_rows = jnp.concatenate(
      [ei[1], TRASH + jnp.arange(pad, dtype=jnp.int32) % N_TRASH]
  ).reshape(N_ROWS, CHUNK)
  ones = jnp.ones((N_NODES, 16), jnp.float32)

  hist = _deg_hist(dst_rows, ones)

  gx = pl.pallas_call(
      _k0_body,
      out_shape=jax.ShapeDtypeStruct((NC, N_NODES, 128), jnp.float32),
      grid=(GRID,),
      in_specs=[_hist_spec(), _row_spec(128)],
      out_specs=_split_spec(128),
  )(hist, x)

  px = _agg_esplit(gx, src_rows, dst_rows)

  g1 = pl.pallas_call(
      _k1_body,
      out_shape=jax.ShapeDtypeStruct((NC, N_NODES, 128), jnp.float32),
      grid=(GRID,),
      in_specs=[_hist_spec(), _split_spec(128), _row_spec(128),
                _full((128, 256)), _full((1, 256))],
      out_specs=_split_spec(128),
  )(hist, px, x, W1, b1.reshape(1, -1))

  agg1 = _agg_fsplit(g1, src_rows, dst_rows)

  g3 = pl.pallas_call(
      _k2_body,
      out_shape=jax.ShapeDtypeStruct((NC, N_NODES, 128), jnp.float32),
      grid=(GRID,),
      in_specs=[_hist_spec(), _split_spec(128),
                _full((256, 256)), _full((1, 256)), _full((256, 40))],
      out_specs=_split_spec(128),
  )(hist, agg1, W2, b2.reshape(1, -1), W3)

  p3 = _agg_esplit(g3, src_rows, dst_rows)

  out = pl.pallas_call(
      _k3_body,
      out_shape=jax.ShapeDtypeStruct((N_NODES, 40), jnp.float32),
      grid=(GRID,),
      in_specs=[_hist_spec(), _split_spec(128), _split_spec(128),
                _full((1, 40))],
      out_specs=_row_spec(40),
  )(hist, p3, g3, b3.reshape(1, -1))

  return out


# speedup vs baseline: 1.0446x; 1.0446x over previous
import jax
import jax.numpy as jnp
from jax import lax
from jax.experimental import pallas as pl
from jax.experimental.pallas import tpu as pltpu
from jax.experimental.pallas import tpu_sc as plsc

N_NODES = 10000
N_EDGES = 320000
CHUNK = 128
NC = 2
NS = 16
E_PAD = 327680
N_ROWS = E_PAD // CHUNK
TRASH = N_NODES
N_TRASH = 128
N_ACC = 10128
BLK = 1000
GRID = N_NODES // BLK

_VMESH = plsc.VectorSubcoreMesh(
    core_axis_name="c", subcore_axis_name="s", num_cores=NC, num_subcores=NS
)

R_MAIN = 632
R_LAST = N_NODES - 15 * R_MAIN

IB = 16


def _copy_rows(src_at, dst_at, s):
  @pl.when(s < NS - 1)
  def _():
    r0 = s * R_MAIN
    pltpu.sync_copy(src_at.at[pl.ds(r0, R_MAIN)], dst_at.at[pl.ds(r0, R_MAIN)])

  @pl.when(s == NS - 1)
  def _():
    r0 = (NS - 1) * R_MAIN
    pltpu.sync_copy(src_at.at[pl.ds(r0, R_LAST)], dst_at.at[pl.ds(r0, R_LAST)])


def _gather_scatter_block(g_at, accum, src_v, dst_v, bufs, sems):
  descs = [None, None]
  descs[0] = pltpu.async_copy(g_at.at[src_v.at[0]], bufs[0], sems[0])
  for k in range(IB):
    if k + 1 < IB:
      nxt = (k + 1) % 2
      descs[nxt] = pltpu.async_copy(g_at.at[src_v.at[k + 1]], bufs[nxt],
                                    sems[nxt])
    descs[k % 2].wait()
    pltpu.sync_copy(bufs[k % 2], accum.at[dst_v.at[k]], add=True)


def _deg_hist(dst_rows, ones_hbm):
  rows_pt = N_ROWS // (NC * NS)

  def body(dst_hbm, ones_r, hist_hbm, accum, idx_v, ones_v):
    c = lax.axis_index("c")
    s = lax.axis_index("s")
    _copy_rows(ones_r, accum, s)
    pltpu.sync_copy(ones_r.at[pl.ds(0, CHUNK)], ones_v)
    plsc.subcore_barrier()

    @pl.loop(0, rows_pt // IB)
    def _(j):
      row0 = (c * NS + s) * rows_pt + j * IB
      pltpu.sync_copy(dst_hbm.at[pl.ds(row0, IB)], idx_v)

      @pl.loop(0, IB)
      def _(k):
        pltpu.sync_copy(ones_v, accum.at[idx_v.at[k]], add=True)

    plsc.subcore_barrier()
    _copy_rows(accum, hist_hbm.at[c], s)

  kern = pl.kernel(
      body,
      out_type=jax.ShapeDtypeStruct((NC, N_NODES, 16), jnp.float32),
      mesh=_VMESH,
      scratch_types=[
          pltpu.VMEM_SHARED((N_ACC, 16), jnp.float32),
          pltpu.VMEM((IB, CHUNK), jnp.int32),
          pltpu.VMEM((CHUNK, 16), jnp.float32),
      ],
  )
  return kern(dst_rows, ones_hbm)


def _agg_fsplit(g2, src_rows, dst_rows):
  rows_pt = N_ROWS // NS

  def body(g_hbm, src_hbm, dst_hbm, agg_hbm, accum,
           src_v, dst_v, rv0, rv1, sem0, sem1):
    c = lax.axis_index("c")
    s = lax.axis_index("s")
    _copy_rows(g_hbm.at[c], accum, s)
    plsc.subcore_barrier()

    @pl.loop(0, rows_pt // IB)
    def _(j):
      row0 = s * rows_pt + j * IB
      pltpu.sync_copy(src_hbm.at[pl.ds(row0, IB)], src_v)
      pltpu.sync_copy(dst_hbm.at[pl.ds(row0, IB)], dst_v)
      _gather_scatter_block(g_hbm.at[c], accum, src_v, dst_v,
                            (rv0, rv1), (sem0, sem1))

    plsc.subcore_barrier()
    _copy_rows(accum, agg_hbm.at[c], s)

  kern = pl.kernel(
      body,
      out_type=jax.ShapeDtypeStruct((NC, N_NODES, 128), jnp.float32),
      mesh=_VMESH,
      scratch_types=[
          pltpu.VMEM_SHARED((N_ACC, 128), jnp.float32),
          pltpu.VMEM((IB, CHUNK), jnp.int32),
          pltpu.VMEM((IB, CHUNK), jnp.int32),
          pltpu.VMEM((CHUNK, 128), jnp.float32),
          pltpu.VMEM((CHUNK, 128), jnp.float32),
          pltpu.SemaphoreType.DMA,
          pltpu.SemaphoreType.DMA,
      ],
  )
  return kern(g2, src_rows, dst_rows)


def _agg_esplit(g, src_rows, dst_rows):
  rows_pt = N_ROWS // (NC * NS)

  def body(g_hbm, src_hbm, dst_hbm, p_hbm, accum,
           src_v, dst_v, rv0, rv1, sem0, sem1):
    c = lax.axis_index("c")
    s = lax.axis_index("s")
    _copy_rows(g_hbm, accum, s)
    plsc.subcore_barrier()

    @pl.loop(0, rows_pt // IB)
    def _(j):
      row0 = ((1 - c) * NS + s) * rows_pt + j * IB
      pltpu.sync_copy(src_hbm.at[pl.ds(row0, IB)], src_v)
      pltpu.sync_copy(dst_hbm.at[pl.ds(row0, IB)], dst_v)
      _gather_scatter_block(g_hbm, accum, src_v, dst_v,
                            (rv0, rv1), (sem0, sem1))

    plsc.subcore_barrier()
    _copy_rows(accum, p_hbm.at[c], s)

  kern = pl.kernel(
      body,
      out_type=jax.ShapeDtypeStruct((NC, N_NODES, 128), jnp.float32),
      mesh=_VMESH,
      scratch_types=[
          pltpu.VMEM_SHARED((N_ACC, 128), jnp.float32),
          pltpu.VMEM((IB, CHUNK), jnp.int32),
          pltpu.VMEM((IB, CHUNK), jnp.int32),
          pltpu.VMEM((CHUNK, 128), jnp.float32),
          pltpu.VMEM((CHUNK, 128), jnp.float32),
          pltpu.SemaphoreType.DMA,
          pltpu.SemaphoreType.DMA,
      ],
  )
  return kern(g, src_rows, dst_rows)


def _dinv(hist_ref):
  deg = hist_ref[0, :, 0:1] + hist_ref[1, :, 0:1] - 1.0
  return lax.rsqrt(deg)


def _k0_body(hist_ref, x_ref, gx_ref):
  gx_ref[...] = x_ref[...] * _dinv(hist_ref)


def _k1_body(hist_ref, px_ref, x_ref, w_ref, b_ref, g1_ref):
  dinv = _dinv(hist_ref)
  a = (px_ref[0] + px_ref[1] - x_ref[...] * dinv) * dinv
  h = jnp.dot(a, w_ref[...], preferred_element_type=jnp.float32) + b_ref[...]
  h = jnp.maximum(h, 0.0)
  g1_ref[0, :, :] = h[:, :128] * dinv
  g1_ref[1, :, :] = h[:, 128:] * dinv


def _k2_body(hist_ref, agg_ref, w2_ref, b2_ref, w3_ref, g3_ref):
  dinv = _dinv(hist_ref)
  a = jnp.concatenate([agg_ref[0], agg_ref[1]], axis=1) * dinv
  h2 = jnp.dot(a, w2_ref[...], preferred_element_type=jnp.float32) + b2_ref[...]
  h2 = jnp.maximum(h2, 0.0)
  m = jnp.dot(h2, w3_ref[...], preferred_element_type=jnp.float32) * dinv
  g3_ref[...] = jnp.concatenate(
      [m, jnp.zeros((BLK, 128 - 40), jnp.float32)], axis=1)


def _k3_body(hist_ref, p3_ref, g3_ref, b3_ref, out_ref):
  dinv = _dinv(hist_ref)
  a = (p3_ref[0] + p3_ref[1] - g3_ref[...]) * dinv
  out_ref[...] = a[:, :40] + b3_ref[...]


def _hist_spec():
  return pl.BlockSpec((NC, BLK, 16), lambda i: (0, i, 0))


def _split_spec(f2):
  return pl.BlockSpec((NC, BLK, f2), lambda i: (0, i, 0))


def _row_spec(width):
  return pl.BlockSpec((BLK, width), lambda i: (i, 0))


def _full(shape):
  return pl.BlockSpec(shape, lambda i: tuple(0 for _ in shape))


def kernel(x, edge_index, W1, b1, W2, b2, W3, b3):
  ei = edge_index.astype(jnp.int32)
  pad = E_PAD - N_EDGES
  src_rows = jnp.concatenate(
      [ei[0], jnp.zeros((pad,), jnp.int32)]).reshape(N_ROWS, CHUNK)
  dst_rows = jnp.concatenate(
      [ei[1], TRASH + jnp.arange(pad, dtype=jnp.int32) % N_TRASH]
  ).reshape(N_ROWS, CHUNK)
  ones = jnp.ones((N_NODES, 16), jnp.float32)

  hist = _deg_hist(dst_rows, ones)

  gx = pl.pallas_call(
      _k0_body,
      out_shape=jax.ShapeDtypeStruct((N_NODES, 128), jnp.float32),
      grid=(GRID,),
      in_specs=[_hist_spec(), _row_spec(128)],
      out_specs=_row_spec(128),
  )(hist, x)

  px = _agg_esplit(gx, src_rows, dst_rows)

  g1 = pl.pallas_call(
      _k1_body,
      out_shape=jax.ShapeDtypeStruct((NC, N_NODES, 128), jnp.float32),
      grid=(GRID,),
      in_specs=[_hist_spec(), _split_spec(128), _row_spec(128),
                _full((128, 256)), _full((1, 256))],
      out_specs=_split_spec(128),
  )(hist, px, x, W1, b1.reshape(1, -1))

  agg1 = _agg_fsplit(g1, src_rows, dst_rows)

  g3 = pl.pallas_call(
      _k2_body,
      out_shape=jax.ShapeDtypeStruct((N_NODES, 128), jnp.float32),
      grid=(GRID,),
      in_specs=[_hist_spec(), _split_spec(128),
                _full((256, 256)), _full((1, 256)), _full((256, 40))],
      out_specs=_row_spec(128),
  )(hist, agg1, W2, b2.reshape(1, -1), W3)

  p3 = _agg_esplit(g3, src_rows, dst_rows)

  out = pl.pallas_call(
      _k3_body,
      out_shape=jax.ShapeDtypeStruct((N_NODES, 40), jnp.float32),
      grid=(GRID,),
      in_specs=[_hist_spec(), _split_spec(128), _row_spec(128),
                _full((1, 40))],
      out_specs=_row_spec(40),
  )(hist, p3, g3, b3.reshape(1, -1))

  return out


# speedup vs baseline: 2.9159x; 2.7914x over previous
import jax
import jax.numpy as jnp
from jax import lax
from jax.experimental import pallas as pl
from jax.experimental.pallas import tpu as pltpu
from jax.experimental.pallas import tpu_sc as plsc

N_NODES = 10000
N_EDGES = 320000
CHUNK = 128
NC = 2
NS = 16
E_PAD = 327680
N_ROWS = E_PAD // CHUNK
TRASH = N_NODES
N_TRASH = 128
N_ACC = 10128
BLK = 1000
GRID = N_NODES // BLK

_VMESH = plsc.VectorSubcoreMesh(
    core_axis_name="c", subcore_axis_name="s", num_cores=NC, num_subcores=NS
)

R_MAIN = 632
R_LAST = N_NODES - 15 * R_MAIN

IB = 16


def _copy_rows(src_at, dst_at, s):
  @pl.when(s < NS - 1)
  def _():
    r0 = s * R_MAIN
    pltpu.sync_copy(src_at.at[pl.ds(r0, R_MAIN)], dst_at.at[pl.ds(r0, R_MAIN)])

  @pl.when(s == NS - 1)
  def _():
    r0 = (NS - 1) * R_MAIN
    pltpu.sync_copy(src_at.at[pl.ds(r0, R_LAST)], dst_at.at[pl.ds(r0, R_LAST)])


def _gather_scatter_block(g_at, accum, src_v, dst_v, bufs, sems):
  descs = [None, None]
  descs[0] = pltpu.async_copy(g_at.at[src_v.at[0]], bufs[0], sems[0])
  for k in range(IB):
    if k + 1 < IB:
      nxt = (k + 1) % 2
      descs[nxt] = pltpu.async_copy(g_at.at[src_v.at[k + 1]], bufs[nxt],
                                    sems[nxt])
    descs[k % 2].wait()
    pltpu.sync_copy(bufs[k % 2], accum.at[dst_v.at[k]], add=True)


def _deg_hist(dst_rows, ones_hbm):
  rows_pt = N_ROWS // (NC * NS)

  def body(dst_hbm, ones_r, hist_hbm, accum, idx_v, ones_v):
    c = lax.axis_index("c")
    s = lax.axis_index("s")
    _copy_rows(ones_r, accum, s)
    pltpu.sync_copy(ones_r.at[pl.ds(0, CHUNK)], ones_v)
    plsc.subcore_barrier()

    @pl.loop(0, rows_pt // IB)
    def _(j):
      row0 = (c * NS + s) * rows_pt + j * IB
      pltpu.sync_copy(dst_hbm.at[pl.ds(row0, IB)], idx_v)

      @pl.loop(0, IB)
      def _(k):
        pltpu.sync_copy(ones_v, accum.at[idx_v.at[k]], add=True)

    plsc.subcore_barrier()
    _copy_rows(accum, hist_hbm.at[c], s)

  kern = pl.kernel(
      body,
      out_type=jax.ShapeDtypeStruct((NC, N_NODES, 16), jnp.float32),
      mesh=_VMESH,
      scratch_types=[
          pltpu.VMEM_SHARED((N_ACC, 16), jnp.float32),
          pltpu.VMEM((IB, CHUNK), jnp.int32),
          pltpu.VMEM((CHUNK, 16), jnp.float32),
      ],
  )
  return kern(dst_rows, ones_hbm)


def _agg_fsplit(g2, src_rows, dst_rows):
  rows_pt = N_ROWS // NS

  def body(g_hbm, src_hbm, dst_hbm, agg_hbm, accum,
           src_v, dst_v, rv0, rv1, sem0, sem1):
    c = lax.axis_index("c")
    s = lax.axis_index("s")
    _copy_rows(g_hbm.at[c], accum, s)
    plsc.subcore_barrier()

    @pl.loop(0, rows_pt // IB)
    def _(j):
      row0 = s * rows_pt + j * IB
      pltpu.sync_copy(src_hbm.at[pl.ds(row0, IB)], src_v)
      pltpu.sync_copy(dst_hbm.at[pl.ds(row0, IB)], dst_v)
      _gather_scatter_block(g_hbm.at[c], accum, src_v, dst_v,
                            (rv0, rv1), (sem0, sem1))

    plsc.subcore_barrier()
    _copy_rows(accum, agg_hbm.at[c], s)

  kern = pl.kernel(
      body,
      out_type=jax.ShapeDtypeStruct((NC, N_NODES, 128), jnp.float32),
      mesh=_VMESH,
      scratch_types=[
          pltpu.VMEM_SHARED((N_ACC, 128), jnp.float32),
          pltpu.VMEM((IB, CHUNK), jnp.int32),
          pltpu.VMEM((IB, CHUNK), jnp.int32),
          pltpu.VMEM((CHUNK, 128), jnp.float32),
          pltpu.VMEM((CHUNK, 128), jnp.float32),
          pltpu.SemaphoreType.DMA,
          pltpu.SemaphoreType.DMA,
      ],
  )
  return kern(g2, src_rows, dst_rows)


def _agg_esplit(g, src_rows, dst_rows):
  rows_pt = N_ROWS // (NC * NS)

  def body(g_hbm, src_hbm, dst_hbm, p_hbm, accum,
           src_v, dst_v, rv0, rv1, sem0, sem1):
    c = lax.axis_index("c")
    s = lax.axis_index("s")
    _copy_rows(g_hbm, accum, s)
    plsc.subcore_barrier()

    @pl.loop(0, rows_pt // IB)
    def _(j):
      row0 = ((1 - c) * NS + s) * rows_pt + j * IB
      pltpu.sync_copy(src_hbm.at[pl.ds(row0, IB)], src_v)
      pltpu.sync_copy(dst_hbm.at[pl.ds(row0, IB)], dst_v)
      _gather_scatter_block(g_hbm, accum, src_v, dst_v,
                            (rv0, rv1), (sem0, sem1))

    plsc.subcore_barrier()
    _copy_rows(accum, p_hbm.at[c], s)

  kern = pl.kernel(
      body,
      out_type=jax.ShapeDtypeStruct((NC, N_NODES, 128), jnp.float32),
      mesh=_VMESH,
      scratch_types=[
          pltpu.VMEM_SHARED((N_ACC, 128), jnp.float32),
          pltpu.VMEM((IB, CHUNK), jnp.int32),
          pltpu.VMEM((IB, CHUNK), jnp.int32),
          pltpu.VMEM((CHUNK, 128), jnp.float32),
          pltpu.VMEM((CHUNK, 128), jnp.float32),
          pltpu.SemaphoreType.DMA,
          pltpu.SemaphoreType.DMA,
      ],
  )
  return kern(g, src_rows, dst_rows)


def _dinv(hist_ref):
  deg = hist_ref[0, :, 0:1] + hist_ref[1, :, 0:1] - 1.0
  return lax.rsqrt(deg)


def _k0_body(hist_ref, x_ref, gx_ref):
  gx_ref[...] = x_ref[...] * _dinv(hist_ref)


def _k1_body(hist_ref, px_ref, x_ref, w_ref, b_ref, g1_ref):
  dinv = _dinv(hist_ref)
  a = (px_ref[0] + px_ref[1] - x_ref[...] * dinv) * dinv
  h = jnp.dot(a, w_ref[...], preferred_element_type=jnp.float32) + b_ref[...]
  h = jnp.maximum(h, 0.0)
  g1_ref[0, :, :] = h[:, :128] * dinv
  g1_ref[1, :, :] = h[:, 128:] * dinv


def _k2_body(hist_ref, agg_ref, w2_ref, b2_ref, w3_ref, g3_ref):
  dinv = _dinv(hist_ref)
  a = jnp.concatenate([agg_ref[0], agg_ref[1]], axis=1) * dinv
  h2 = jnp.dot(a, w2_ref[...], preferred_element_type=jnp.float32) + b2_ref[...]
  h2 = jnp.maximum(h2, 0.0)
  m = jnp.dot(h2, w3_ref[...], preferred_element_type=jnp.float32) * dinv
  g3_ref[...] = jnp.concatenate(
      [m, jnp.zeros((BLK, 128 - 40), jnp.float32)], axis=1)


def _k3_body(hist_ref, p3_ref, g3_ref, b3_ref, out_ref):
  dinv = _dinv(hist_ref)
  a = (p3_ref[0] + p3_ref[1] - g3_ref[...]) * dinv
  out_ref[...] = a[:, :40] + b3_ref[...]


def _hist_spec():
  return pl.BlockSpec((NC, BLK, 16), lambda i: (0, i, 0))


def _split_spec(f2):
  return pl.BlockSpec((NC, BLK, f2), lambda i: (0, i, 0))


def _row_spec(width):
  return pl.BlockSpec((BLK, width), lambda i: (i, 0))


def _full(shape):
  return pl.BlockSpec(shape, lambda i: tuple(0 for _ in shape))


def kernel(x, edge_index, W1, b1, W2, b2, W3, b3):
  ei = edge_index.astype(jnp.int32)
  pad = E_PAD - N_EDGES
  src_rows = jnp.concatenate(
      [ei[0], jnp.arange(pad, dtype=jnp.int32) % N_NODES]
  ).reshape(N_ROWS, CHUNK)
  dst_rows = jnp.concatenate(
      [ei[1], TRASH + jnp.arange(pad, dtype=jnp.int32) % N_TRASH]
  ).reshape(N_ROWS, CHUNK)
  ones = jnp.ones((N_NODES, 16), jnp.float32)

  hist = _deg_hist(dst_rows, ones)

  gx = pl.pallas_call(
      _k0_body,
      out_shape=jax.ShapeDtypeStruct((N_NODES, 128), jnp.float32),
      grid=(GRID,),
      in_specs=[_hist_spec(), _row_spec(128)],
      out_specs=_row_spec(128),
  )(hist, x)

  px = _agg_esplit(gx, src_rows, dst_rows)

  g1 = pl.pallas_call(
      _k1_body,
      out_shape=jax.ShapeDtypeStruct((NC, N_NODES, 128), jnp.float32),
      grid=(GRID,),
      in_specs=[_hist_spec(), _split_spec(128), _row_spec(128),
                _full((128, 256)), _full((1, 256))],
      out_specs=_split_spec(128),
  )(hist, px, x, W1, b1.reshape(1, -1))

  agg1 = _agg_fsplit(g1, src_rows, dst_rows)

  g3 = pl.pallas_call(
      _k2_body,
      out_shape=jax.ShapeDtypeStruct((N_NODES, 128), jnp.float32),
      grid=(GRID,),
      in_specs=[_hist_spec(), _split_spec(128),
                _full((256, 256)), _full((1, 256)), _full((256, 40))],
      out_specs=_row_spec(128),
  )(hist, agg1, W2, b2.reshape(1, -1), W3)

  p3 = _agg_esplit(g3, src_rows, dst_rows)

  out = pl.pallas_call(
      _k3_body,
      out_shape=jax.ShapeDtypeStruct((N_NODES, 40), jnp.float32),
      grid=(GRID,),
      in_specs=[_hist_spec(), _split_spec(128), _row_spec(128),
                _full((1, 40))],
      out_specs=_row_spec(40),
  )(hist, p3, g3, b3.reshape(1, -1))

  return out


# speedup vs baseline: 3.0770x; 1.0553x over previous
import jax
import jax.numpy as jnp
from jax import lax
from jax.experimental import pallas as pl
from jax.experimental.pallas import tpu as pltpu
from jax.experimental.pallas import tpu_sc as plsc

N_NODES = 10000
N_EDGES = 320000
CHUNK = 128
NC = 2
NS = 16
E_PAD = 327680
N_ROWS = E_PAD // CHUNK
TRASH = N_NODES
N_TRASH = 128
N_ACC = 10128
BLK = 1000
GRID = N_NODES // BLK

_VMESH = plsc.VectorSubcoreMesh(
    core_axis_name="c", subcore_axis_name="s", num_cores=NC, num_subcores=NS
)

R_MAIN = 632
R_LAST = N_NODES - 15 * R_MAIN

IB = 40


def _copy_rows(src_at, dst_at, s):
  @pl.when(s < NS - 1)
  def _():
    r0 = s * R_MAIN
    pltpu.sync_copy(src_at.at[pl.ds(r0, R_MAIN)], dst_at.at[pl.ds(r0, R_MAIN)])

  @pl.when(s == NS - 1)
  def _():
    r0 = (NS - 1) * R_MAIN
    pltpu.sync_copy(src_at.at[pl.ds(r0, R_LAST)], dst_at.at[pl.ds(r0, R_LAST)])


def _gather_scatter_block(g_at, accum, src_v, dst_v, bufs, sems):
  descs = [None, None]
  descs[0] = pltpu.async_copy(g_at.at[src_v.at[0]], bufs[0], sems[0])
  for k in range(IB):
    if k + 1 < IB:
      nxt = (k + 1) % 2
      descs[nxt] = pltpu.async_copy(g_at.at[src_v.at[k + 1]], bufs[nxt],
                                    sems[nxt])
    descs[k % 2].wait()
    pltpu.sync_copy(bufs[k % 2], accum.at[dst_v.at[k]], add=True)


def _deg_hist(dst_rows, ones_hbm):
  rows_pt = N_ROWS // (NC * NS)

  def body(dst_hbm, ones_r, hist_hbm, accum, idx_v, ones_v):
    c = lax.axis_index("c")
    s = lax.axis_index("s")
    _copy_rows(ones_r, accum, s)
    pltpu.sync_copy(ones_r.at[pl.ds(0, CHUNK)], ones_v)
    plsc.subcore_barrier()

    @pl.loop(0, rows_pt // IB)
    def _(j):
      row0 = (c * NS + s) * rows_pt + j * IB
      pltpu.sync_copy(dst_hbm.at[pl.ds(row0, IB)], idx_v)

      @pl.loop(0, IB)
      def _(k):
        pltpu.sync_copy(ones_v, accum.at[idx_v.at[k]], add=True)

    plsc.subcore_barrier()
    _copy_rows(accum, hist_hbm.at[c], s)

  kern = pl.kernel(
      body,
      out_type=jax.ShapeDtypeStruct((NC, N_NODES, 16), jnp.float32),
      mesh=_VMESH,
      scratch_types=[
          pltpu.VMEM_SHARED((N_ACC, 16), jnp.float32),
          pltpu.VMEM((IB, CHUNK), jnp.int32),
          pltpu.VMEM((CHUNK, 16), jnp.float32),
      ],
  )
  return kern(dst_rows, ones_hbm)


def _agg_fsplit(g2, src_rows, dst_rows):
  rows_pt = N_ROWS // NS

  def body(g_hbm, src_hbm, dst_hbm, agg_hbm, accum,
           src_v, dst_v, rv0, rv1, sem0, sem1):
    c = lax.axis_index("c")
    s = lax.axis_index("s")
    _copy_rows(g_hbm.at[c], accum, s)
    plsc.subcore_barrier()

    @pl.loop(0, rows_pt // IB)
    def _(j):
      row0 = s * rows_pt + j * IB
      pltpu.sync_copy(src_hbm.at[pl.ds(row0, IB)], src_v)
      pltpu.sync_copy(dst_hbm.at[pl.ds(row0, IB)], dst_v)
      _gather_scatter_block(g_hbm.at[c], accum, src_v, dst_v,
                            (rv0, rv1), (sem0, sem1))

    plsc.subcore_barrier()
    _copy_rows(accum, agg_hbm.at[c], s)

  kern = pl.kernel(
      body,
      out_type=jax.ShapeDtypeStruct((NC, N_NODES, 128), jnp.float32),
      mesh=_VMESH,
      scratch_types=[
          pltpu.VMEM_SHARED((N_ACC, 128), jnp.float32),
          pltpu.VMEM((IB, CHUNK), jnp.int32),
          pltpu.VMEM((IB, CHUNK), jnp.int32),
          pltpu.VMEM((CHUNK, 128), jnp.float32),
          pltpu.VMEM((CHUNK, 128), jnp.float32),
          pltpu.SemaphoreType.DMA,
          pltpu.SemaphoreType.DMA,
      ],
  )
  return kern(g2, src_rows, dst_rows)


def _agg_esplit(g, src_rows, dst_rows):
  rows_pt = N_ROWS // (NC * NS)

  def body(g_hbm, src_hbm, dst_hbm, p_hbm, accum,
           src_v, dst_v, rv0, rv1, sem0, sem1):
    c = lax.axis_index("c")
    s = lax.axis_index("s")
    _copy_rows(g_hbm, accum, s)
    plsc.subcore_barrier()

    @pl.loop(0, rows_pt // IB)
    def _(j):
      row0 = ((1 - c) * NS + s) * rows_pt + j * IB
      pltpu.sync_copy(src_hbm.at[pl.ds(row0, IB)], src_v)
      pltpu.sync_copy(dst_hbm.at[pl.ds(row0, IB)], dst_v)
      _gather_scatter_block(g_hbm, accum, src_v, dst_v,
                            (rv0, rv1), (sem0, sem1))

    plsc.subcore_barrier()
    _copy_rows(accum, p_hbm.at[c], s)

  kern = pl.kernel(
      body,
      out_type=jax.ShapeDtypeStruct((NC, N_NODES, 128), jnp.float32),
      mesh=_VMESH,
      scratch_types=[
          pltpu.VMEM_SHARED((N_ACC, 128), jnp.float32),
          pltpu.VMEM((IB, CHUNK), jnp.int32),
          pltpu.VMEM((IB, CHUNK), jnp.int32),
          pltpu.VMEM((CHUNK, 128), jnp.float32),
          pltpu.VMEM((CHUNK, 128), jnp.float32),
          pltpu.SemaphoreType.DMA,
          pltpu.SemaphoreType.DMA,
      ],
  )
  return kern(g, src_rows, dst_rows)


def _dinv(hist_ref):
  deg = hist_ref[0, :, 0:1] + hist_ref[1, :, 0:1] - 1.0
  return lax.rsqrt(deg)


def _k0_body(hist_ref, x_ref, gx_ref):
  gx_ref[...] = x_ref[...] * _dinv(hist_ref)


def _k1_body(hist_ref, px_ref, x_ref, w_ref, b_ref, g1_ref):
  dinv = _dinv(hist_ref)
  a = (px_ref[0] + px_ref[1] - x_ref[...] * dinv) * dinv
  h = jnp.dot(a, w_ref[...], preferred_element_type=jnp.float32) + b_ref[...]
  h = jnp.maximum(h, 0.0)
  g1_ref[0, :, :] = h[:, :128] * dinv
  g1_ref[1, :, :] = h[:, 128:] * dinv


def _k2_body(hist_ref, agg_ref, w2_ref, b2_ref, w3_ref, g3_ref):
  dinv = _dinv(hist_ref)
  a = jnp.concatenate([agg_ref[0], agg_ref[1]], axis=1) * dinv
  h2 = jnp.dot(a, w2_ref[...], preferred_element_type=jnp.float32) + b2_ref[...]
  h2 = jnp.maximum(h2, 0.0)
  m = jnp.dot(h2, w3_ref[...], preferred_element_type=jnp.float32) * dinv
  g3_ref[...] = jnp.concatenate(
      [m, jnp.zeros((BLK, 128 - 40), jnp.float32)], axis=1)


def _k3_body(hist_ref, p3_ref, g3_ref, b3_ref, out_ref):
  dinv = _dinv(hist_ref)
  a = (p3_ref[0] + p3_ref[1] - g3_ref[...]) * dinv
  out_ref[...] = a[:, :40] + b3_ref[...]


def _hist_spec():
  return pl.BlockSpec((NC, BLK, 16), lambda i: (0, i, 0))


def _split_spec(f2):
  return pl.BlockSpec((NC, BLK, f2), lambda i: (0, i, 0))


def _row_spec(width):
  return pl.BlockSpec((BLK, width), lambda i: (i, 0))


def _full(shape):
  return pl.BlockSpec(shape, lambda i: tuple(0 for _ in shape))


def kernel(x, edge_index, W1, b1, W2, b2, W3, b3):
  ei = edge_index.astype(jnp.int32)
  pad = E_PAD - N_EDGES
  src_rows = jnp.concatenate(
      [ei[0], jnp.arange(pad, dtype=jnp.int32) % N_NODES]
  ).reshape(N_ROWS, CHUNK)
  dst_rows = jnp.concatenate(
      [ei[1], TRASH + jnp.arange(pad, dtype=jnp.int32) % N_TRASH]
  ).reshape(N_ROWS, CHUNK)
  ones = jnp.ones((N_NODES, 16), jnp.float32)

  hist = _deg_hist(dst_rows, ones)

  gx = pl.pallas_call(
      _k0_body,
      out_shape=jax.ShapeDtypeStruct((N_NODES, 128), jnp.float32),
      grid=(GRID,),
      in_specs=[_hist_spec(), _row_spec(128)],
      out_specs=_row_spec(128),
  )(hist, x)

  px = _agg_esplit(gx, src_rows, dst_rows)

  g1 = pl.pallas_call(
      _k1_body,
      out_shape=jax.ShapeDtypeStruct((NC, N_NODES, 128), jnp.float32),
      grid=(GRID,),
      in_specs=[_hist_spec(), _split_spec(128), _row_spec(128),
                _full((128, 256)), _full((1, 256))],
      out_specs=_split_spec(128),
  )(hist, px, x, W1, b1.reshape(1, -1))

  agg1 = _agg_fsplit(g1, src_rows, dst_rows)

  g3 = pl.pallas_call(
      _k2_body,
      out_shape=jax.ShapeDtypeStruct((N_NODES, 128), jnp.float32),
      grid=(GRID,),
      in_specs=[_hist_spec(), _split_spec(128),
                _full((256, 256)), _full((1, 256)), _full((256, 40))],
      out_specs=_row_spec(128),
  )(hist, agg1, W2, b2.reshape(1, -1), W3)

  p3 = _agg_esplit(g3, src_rows, dst_rows)

  out = pl.pallas_call(
      _k3_body,
      out_shape=jax.ShapeDtypeStruct((N_NODES, 40), jnp.float32),
      grid=(GRID,),
      in_specs=[_hist_spec(), _split_spec(128), _row_spec(128),
                _full((1, 40))],
      out_specs=_row_spec(40),
  )(hist, p3, g3, b3.reshape(1, -1))

  return out
